# baseline (speedup 1.0000x reference)
import functools

import jax
import jax.numpy as jnp
from jax import lax
from jax.experimental import pallas as pl
from jax.experimental.pallas import tpu as pltpu
from jax.experimental.pallas import tpu_sc as plsc

N = 10000
E = 20000
NNZ = 320000
D_IN = 128
D_HID = 256
CW = 64
NCHUNK = D_HID // CW
NC = 2
NS = 16
EPT = E // NS
VPT = N // NS
BS = 4000
NB = NNZ // BS
BC = 512
FLUSH_AT = BC - 16


def _matmul(X, W):
    def body(x_ref, w_ref, o_ref):
        o_ref[...] = lax.dot_general(
            x_ref[...], w_ref[...], (((1,), (1,)), ((), ())),
            preferred_element_type=jnp.float32)

    return pl.pallas_call(
        body,
        grid=(25,),
        in_specs=[pl.BlockSpec((400, D_IN), lambda i: (i, 0)),
                  pl.BlockSpec((D_HID, D_IN), lambda i: (0, 0))],
        out_specs=pl.BlockSpec((400, D_HID), lambda i: (i, 0)),
        out_shape=jax.ShapeDtypeStruct((N, D_HID), jnp.float32),
    )(X, W)


def _epilogue(Xw, Xv):
    def body(a_ref, b_ref, o_ref):
        s = a_ref[...] + b_ref[...]
        rn2 = jnp.sum(s * s, axis=1, keepdims=True)
        scale = jnp.where(rn2 > 0, lax.rsqrt(rn2), 0.0)
        o_ref[...] = s * scale

    return pl.pallas_call(
        body,
        grid=(25,),
        in_specs=[pl.BlockSpec((400, D_HID), lambda i: (i, 0)),
                  pl.BlockSpec((400, D_HID), lambda i: (i, 0))],
        out_specs=pl.BlockSpec((400, D_HID), lambda i: (i, 0)),
        out_shape=jax.ShapeDtypeStruct((N, D_HID), jnp.float32),
    )(Xw, Xv)


def _sc_conv(xwc, vertex, edges):
    mesh = plsc.VectorSubcoreMesh(core_axis_name="c", subcore_axis_name="s",
                                  num_cores=NC, num_subcores=NS)

    @functools.partial(
        pl.kernel,
        out_type=(jax.ShapeDtypeStruct((NCHUNK * N, CW), jnp.float32),
                  jax.ShapeDtypeStruct((NCHUNK * E, CW), jnp.float32)),
        mesh=mesh,
        compiler_params=pltpu.CompilerParams(use_tc_tiling_on_sc=False,
                                            needs_layout_passes=False),
        scratch_types=[
            pltpu.VMEM((EPT, CW), jnp.float32),
            pltpu.VMEM((EPT + 16,), jnp.float32),
            pltpu.VMEM((BC, CW), jnp.float32),
            pltpu.VMEM((BS,), jnp.int32),
            pltpu.VMEM((BS,), jnp.int32),
            pltpu.VMEM((BC,), jnp.int32),
            pltpu.VMEM((BC + 16,), jnp.int32),
            pltpu.SemaphoreType.DMA,
        ],
    )
    def conv(xwc_hbm, vtx_hbm, edg_hbm, xvc_hbm, xec_hbm,
             acc, cnt, rows, vbuf, ebuf, cidx, cacc, sem):
        sid = lax.axis_index("s")
        cid = lax.axis_index("c")

        iota16 = lax.iota(jnp.int32, 16)

        def init_cidx(j, _):
            cidx[pl.ds(j * 16, 16)] = iota16 + j * 16
            return 0
        lax.fori_loop(0, BC // 16, init_cidx, 0)

        zero16 = jnp.zeros((16,), jnp.float32)

        def zero_acc(nrows):
            def zr(r, _):
                for k in range(CW // 16):
                    acc[r, pl.ds(k * 16, 16)] = zero16
                return 0
            lax.fori_loop(0, nrows, zr, 0)

        def zero_cnt():
            def zc(j, _):
                cnt[pl.ds(j * 16, 16)] = zero16
                return 0
            lax.fori_loop(0, (EPT + 16) // 16, zc, 0)

        def do_stage(stage1, c, lo, src_hbm):
            span = EPT if stage1 else VPT
            hi = lo + span
            coff = c * (N if stage1 else E)

            def flush(p):
                pltpu.async_copy(src_hbm.at[cidx], rows, sem).wait()

                def accum(r, _):
                    a = cacc[pl.ds(r, 16)][0]
                    for k in range(CW // 16):
                        sl = pl.ds(k * 16, 16)
                        acc[a, sl] = acc[a, sl] + rows[r, sl]
                    if stage1:
                        base = (a // 16) * 16
                        lane = a - base
                        cv = cnt[pl.ds(base, 16)]
                        cnt[pl.ds(base, 16)] = cv + jnp.where(
                            iota16 == lane, 1.0, 0.0)
                    return 0
                lax.fori_loop(0, p, accum, 0)
                return jnp.int32(0)

            def vec_body(j, ptr):
                ev = ebuf[pl.ds(j * 16, 16)]
                vv = vbuf[pl.ds(j * 16, 16)]
                seg = ev if stage1 else vv
                oth = vv if stage1 else ev
                m = (seg >= lo) & (seg < hi)
                plsc.store_compressed(cacc.at[pl.ds(ptr, 16)], seg - lo,
                                      mask=m)
                plsc.store_compressed(cidx.at[pl.ds(ptr, 16)], oth + coff,
                                      mask=m)
                ptr = ptr + jnp.sum(m.astype(jnp.int32))
                return lax.cond(ptr >= FLUSH_AT, flush, lambda p: p, ptr)

            def batch(b, ptr):
                pltpu.sync_copy(vtx_hbm.at[pl.ds(b * BS, BS)], vbuf)
                pltpu.sync_copy(edg_hbm.at[pl.ds(b * BS, BS)], ebuf)
                return lax.fori_loop(0, BS // 16, vec_body, ptr)

            ptr = lax.fori_loop(0, NB, batch, jnp.int32(0))
            flush(ptr)

        for t in range(2):
            c = cid * 2 + t
            zero_acc(EPT)
            zero_cnt()
            do_stage(True, c, sid * EPT, xwc_hbm)

            def recip(j, _):
                sl = pl.ds(j * 16, 16)
                cnt[sl] = 1.0 / jnp.maximum(cnt[sl], 1.0)
                return 0
            lax.fori_loop(0, (EPT + 16) // 16, recip, 0)

            def norm(e, _):
                inv = cnt[pl.ds(e, 16)][0]
                for k in range(CW // 16):
                    sl = pl.ds(k * 16, 16)
                    acc[e, sl] = acc[e, sl] * inv
                return 0
            lax.fori_loop(0, EPT, norm, 0)
            pltpu.sync_copy(acc, xec_hbm.at[pl.ds(c * E + sid * EPT, EPT)])

        plsc.subcore_barrier()

        for t in range(2):
            c = cid * 2 + t
            zero_acc(VPT)
            do_stage(False, c, sid * VPT, xec_hbm)
            pltpu.sync_copy(acc.at[pl.ds(0, VPT)],
                            xvc_hbm.at[pl.ds(c * N + sid * VPT, VPT)])

    return conv(xwc, vertex, edges)


def kernel(X, vertex, edges, W):
    Xw = _matmul(X, W)
    xwc = Xw.reshape(N, NCHUNK, CW).transpose(1, 0, 2).reshape(NCHUNK * N, CW)
    xvc, _ = _sc_conv(xwc, vertex, edges)
    Xv = xvc.reshape(NCHUNK, N, CW).transpose(1, 0, 2).reshape(N, D_HID)
    return _epilogue(Xw, Xv)


# speedup vs baseline: 1.0072x; 1.0072x over previous
import functools

import jax
import jax.numpy as jnp
from jax import lax
from jax.experimental import pallas as pl
from jax.experimental.pallas import tpu as pltpu
from jax.experimental.pallas import tpu_sc as plsc

N = 10000
E = 20000
NNZ = 320000
D_IN = 128
D_HID = 256
CW = 64
NCHUNK = D_HID // CW
NC = 2
NS = 16
EPT = E // NS
VPT = N // NS
BS = 4000
NB = NNZ // BS
BC = 512
FLUSH_AT = BC - 16


def _matmul(X, W):
    def body(x_ref, w_ref, o_ref, oc_ref):
        r = lax.dot_general(
            x_ref[...], w_ref[...], (((1,), (1,)), ((), ())),
            preferred_element_type=jnp.float32)
        o_ref[...] = r
        oc_ref[...] = r.reshape(400, NCHUNK, CW).transpose(1, 0, 2)

    out, outc = pl.pallas_call(
        body,
        grid=(25,),
        in_specs=[pl.BlockSpec((400, D_IN), lambda i: (i, 0)),
                  pl.BlockSpec((D_HID, D_IN), lambda i: (0, 0))],
        out_specs=[pl.BlockSpec((400, D_HID), lambda i: (i, 0)),
                   pl.BlockSpec((NCHUNK, 400, CW), lambda i: (0, i, 0))],
        out_shape=[jax.ShapeDtypeStruct((N, D_HID), jnp.float32),
                   jax.ShapeDtypeStruct((NCHUNK, N, CW), jnp.float32)],
    )(X, W)
    return out, outc.reshape(NCHUNK * N, CW)


def _epilogue(Xw, Xv):
    def body(a_ref, b_ref, o_ref):
        s = a_ref[...] + b_ref[...]
        rn2 = jnp.sum(s * s, axis=1, keepdims=True)
        scale = jnp.where(rn2 > 0, lax.rsqrt(rn2), 0.0)
        o_ref[...] = s * scale

    return pl.pallas_call(
        body,
        grid=(25,),
        in_specs=[pl.BlockSpec((400, D_HID), lambda i: (i, 0)),
                  pl.BlockSpec((400, D_HID), lambda i: (i, 0))],
        out_specs=pl.BlockSpec((400, D_HID), lambda i: (i, 0)),
        out_shape=jax.ShapeDtypeStruct((N, D_HID), jnp.float32),
    )(Xw, Xv)


def _sc_conv(xwc, vertex, edges):
    mesh = plsc.VectorSubcoreMesh(core_axis_name="c", subcore_axis_name="s",
                                  num_cores=NC, num_subcores=NS)

    @functools.partial(
        pl.kernel,
        out_type=(jax.ShapeDtypeStruct((N, D_HID), jnp.float32),
                  jax.ShapeDtypeStruct((NCHUNK * E, CW), jnp.float32)),
        mesh=mesh,
        compiler_params=pltpu.CompilerParams(use_tc_tiling_on_sc=False,
                                            needs_layout_passes=False),
        scratch_types=[
            pltpu.VMEM((EPT, CW), jnp.float32),
            pltpu.VMEM((EPT + 16,), jnp.float32),
            pltpu.VMEM((BC, CW), jnp.float32),
            pltpu.VMEM((BS,), jnp.int32),
            pltpu.VMEM((BS,), jnp.int32),
            pltpu.VMEM((BC,), jnp.int32),
            pltpu.VMEM((BC + 16,), jnp.int32),
            pltpu.SemaphoreType.DMA,
        ],
    )
    def conv(xwc_hbm, vtx_hbm, edg_hbm, xv_hbm, xec_hbm,
             acc, cnt, rows, vbuf, ebuf, cidx, cacc, sem):
        sid = lax.axis_index("s")
        cid = lax.axis_index("c")

        iota16 = lax.iota(jnp.int32, 16)

        def init_cidx(j, _):
            cidx[pl.ds(j * 16, 16)] = iota16 + j * 16
            return 0
        lax.fori_loop(0, BC // 16, init_cidx, 0)

        zero16 = jnp.zeros((16,), jnp.float32)

        def zero_acc(nrows):
            def zr(r, _):
                for k in range(CW // 16):
                    acc[r, pl.ds(k * 16, 16)] = zero16
                return 0
            lax.fori_loop(0, nrows, zr, 0)

        def zero_cnt():
            def zc(j, _):
                cnt[pl.ds(j * 16, 16)] = zero16
                return 0
            lax.fori_loop(0, (EPT + 16) // 16, zc, 0)

        def do_stage(stage1, c, lo, src_hbm):
            span = EPT if stage1 else VPT
            hi = lo + span
            coff = c * (N if stage1 else E)

            def flush(p):
                pltpu.async_copy(src_hbm.at[cidx], rows, sem).wait()

                def accum(r, _):
                    a = cacc[pl.ds(r, 16)][0]
                    for k in range(CW // 16):
                        sl = pl.ds(k * 16, 16)
                        acc[a, sl] = acc[a, sl] + rows[r, sl]
                    if stage1:
                        base = (a // 16) * 16
                        lane = a - base
                        cv = cnt[pl.ds(base, 16)]
                        cnt[pl.ds(base, 16)] = cv + jnp.where(
                            iota16 == lane, 1.0, 0.0)
                    return 0
                lax.fori_loop(0, p, accum, 0)
                return jnp.int32(0)

            def vec_body(j, ptr):
                ev = ebuf[pl.ds(j * 16, 16)]
                vv = vbuf[pl.ds(j * 16, 16)]
                seg = ev if stage1 else vv
                oth = vv if stage1 else ev
                m = (seg >= lo) & (seg < hi)
                plsc.store_compressed(cacc.at[pl.ds(ptr, 16)], seg - lo,
                                      mask=m)
                plsc.store_compressed(cidx.at[pl.ds(ptr, 16)], oth + coff,
                                      mask=m)
                ptr = ptr + jnp.sum(m.astype(jnp.int32))
                return lax.cond(ptr >= FLUSH_AT, flush, lambda p: p, ptr)

            def batch(b, ptr):
                pltpu.sync_copy(vtx_hbm.at[pl.ds(b * BS, BS)], vbuf)
                pltpu.sync_copy(edg_hbm.at[pl.ds(b * BS, BS)], ebuf)
                return lax.fori_loop(0, BS // 16, vec_body, ptr)

            ptr = lax.fori_loop(0, NB, batch, jnp.int32(0))
            flush(ptr)

        for t in range(2):
            c = cid * 2 + t
            zero_acc(EPT)
            zero_cnt()
            do_stage(True, c, sid * EPT, xwc_hbm)

            def recip(j, _):
                sl = pl.ds(j * 16, 16)
                cnt[sl] = 1.0 / jnp.maximum(cnt[sl], 1.0)
                return 0
            lax.fori_loop(0, (EPT + 16) // 16, recip, 0)

            def norm(e, _):
                inv = cnt[pl.ds(e, 16)][0]
                for k in range(CW // 16):
                    sl = pl.ds(k * 16, 16)
                    acc[e, sl] = acc[e, sl] * inv
                return 0
            lax.fori_loop(0, EPT, norm, 0)
            pltpu.sync_copy(acc, xec_hbm.at[pl.ds(c * E + sid * EPT, EPT)])

        plsc.subcore_barrier()

        for t in range(2):
            c = cid * 2 + t
            zero_acc(VPT)
            do_stage(False, c, sid * VPT, xec_hbm)
            pltpu.sync_copy(acc.at[pl.ds(0, VPT)],
                            xv_hbm.at[pl.ds(sid * VPT, VPT),
                                      pl.ds(c * CW, CW)])

    return conv(xwc, vertex, edges)


def kernel(X, vertex, edges, W):
    Xw, xwc = _matmul(X, W)
    Xv, _ = _sc_conv(xwc, vertex, edges)
    return _epilogue(Xw, Xv)


# speedup vs baseline: 1.8069x; 1.7940x over previous
import functools

import jax
import jax.numpy as jnp
from jax import lax
from jax.experimental import pallas as pl
from jax.experimental.pallas import tpu as pltpu
from jax.experimental.pallas import tpu_sc as plsc

N = 10000
E = 20000
NNZ = 320000
D_IN = 128
D_HID = 256
CW = 64
NCHUNK = D_HID // CW
NC = 2
NS = 16
EPT = E // NS
VPT = N // NS
DUMP = EPT
BS = 1600
NB = NNZ // BS
FL = 2048
LBUFN = 3664
CAP = (NNZ // FL + 2) * FL
RB = 256
S1_SHIFT, S1_MASK = 14, (1 << 14) - 1
S2_SHIFT, S2_MASK = 15, (1 << 15) - 1


def _matmul(X, W):
    def body(x_ref, w_ref, o_ref, oc_ref):
        r = lax.dot_general(
            x_ref[...], w_ref[...], (((1,), (1,)), ((), ())),
            preferred_element_type=jnp.float32)
        o_ref[...] = r
        oc_ref[...] = r.reshape(400, NCHUNK, CW).transpose(1, 0, 2)

    out, outc = pl.pallas_call(
        body,
        grid=(25,),
        in_specs=[pl.BlockSpec((400, D_IN), lambda i: (i, 0)),
                  pl.BlockSpec((D_HID, D_IN), lambda i: (0, 0))],
        out_specs=[pl.BlockSpec((400, D_HID), lambda i: (i, 0)),
                   pl.BlockSpec((NCHUNK, 400, CW), lambda i: (0, i, 0))],
        out_shape=[jax.ShapeDtypeStruct((N, D_HID), jnp.float32),
                   jax.ShapeDtypeStruct((NCHUNK, N, CW), jnp.float32)],
    )(X, W)
    return out, outc.reshape(NCHUNK * N, CW)


def _epilogue(Xw, Xv):
    def body(a_ref, b_ref, o_ref):
        s = a_ref[...] + b_ref[...]
        rn2 = jnp.sum(s * s, axis=1, keepdims=True)
        scale = jnp.where(rn2 > 0, lax.rsqrt(rn2), 0.0)
        o_ref[...] = s * scale

    return pl.pallas_call(
        body,
        grid=(25,),
        in_specs=[pl.BlockSpec((400, D_HID), lambda i: (i, 0)),
                  pl.BlockSpec((400, D_HID), lambda i: (i, 0))],
        out_specs=pl.BlockSpec((400, D_HID), lambda i: (i, 0)),
        out_shape=jax.ShapeDtypeStruct((N, D_HID), jnp.float32),
    )(Xw, Xv)


def _sc_conv(xwc, vertex, edges):
    mesh = plsc.VectorSubcoreMesh(core_axis_name="c", subcore_axis_name="s",
                                  num_cores=NC, num_subcores=NS)

    @functools.partial(
        pl.kernel,
        out_type=(jax.ShapeDtypeStruct((N, D_HID), jnp.float32),
                  jax.ShapeDtypeStruct((NCHUNK * E, CW), jnp.float32),
                  jax.ShapeDtypeStruct((NC * NS * CAP,), jnp.int32)),
        mesh=mesh,
        compiler_params=pltpu.CompilerParams(use_tc_tiling_on_sc=False,
                                             needs_layout_passes=False),
        scratch_types=[
            pltpu.VMEM((EPT + 1, CW), jnp.float32),
            pltpu.VMEM((1280,), jnp.float32),
            pltpu.VMEM((RB, CW), jnp.float32),
            pltpu.VMEM((RB, CW), jnp.float32),
            pltpu.VMEM((BS,), jnp.int32),
            pltpu.VMEM((BS,), jnp.int32),
            pltpu.VMEM((BS,), jnp.int32),
            pltpu.VMEM((BS,), jnp.int32),
            pltpu.VMEM((LBUFN,), jnp.int32),
            pltpu.VMEM((RB,), jnp.int32),
            pltpu.VMEM((RB,), jnp.int32),
            pltpu.VMEM((RB,), jnp.int32),
            pltpu.VMEM((RB,), jnp.int32),
            pltpu.VMEM((RB + 16,), jnp.int32),
            pltpu.VMEM((RB + 16,), jnp.int32),
            pltpu.SemaphoreType.DMA,
            pltpu.SemaphoreType.DMA,
            pltpu.SemaphoreType.DMA,
            pltpu.SemaphoreType.DMA,
            pltpu.SemaphoreType.DMA,
            pltpu.SemaphoreType.DMA,
        ],
    )
    def conv(xwc_hbm, vtx_hbm, edg_hbm, xv_hbm, xec_hbm, list_hbm,
             acc, cnt, rows0, rows1, vb0, vb1, eb0, eb1, lbuf,
             rlb0, rlb1, ci0, ci1, ca0, ca1,
             semi0, semi1, semg0, semg1, seml0, seml1):
        sid = lax.axis_index("s")
        cid = lax.axis_index("c")
        lbase = (cid * NS + sid) * CAP
        iota16 = lax.iota(jnp.int32, 16)
        zero16 = jnp.zeros((16,), jnp.float32)
        rows = (rows0, rows1)
        vb = (vb0, vb1)
        eb = (eb0, eb1)
        rlb = (rlb0, rlb1)
        ci = (ci0, ci1)
        ca = (ca0, ca1)
        semi = (semi0, semi1)
        semg = (semg0, semg1)
        seml = (seml0, seml1)

        def zero_acc():
            @plsc.parallel_loop(0, EPT + 1, unroll=8)
            def _(r):
                for k in range(CW // 16):
                    acc[r, pl.ds(k * 16, 16)] = zero16

        def zero_cnt():
            @plsc.parallel_loop(0, 1280 // 16, unroll=8)
            def _(j):
                cnt[pl.ds(j * 16, 16)] = zero16

        def scan(stage1, lo):
            span = EPT if stage1 else VPT
            hi = lo + span
            shift = S1_SHIFT if stage1 else S2_SHIFT

            def prefetch(b, ph):
                off = b * BS
                pltpu.async_copy(vtx_hbm.at[pl.ds(off, BS)], vb[ph],
                                 semi[ph])
                pltpu.async_copy(edg_hbm.at[pl.ds(off, BS)], eb[ph],
                                 semi[ph])

            prefetch(0, 0)

            def flush_if_full(carry):
                lptr, app = carry

                def do_flush(args):
                    lptr, app = args
                    off = pl.multiple_of(lbase + app, FL)
                    pltpu.sync_copy(lbuf.at[pl.ds(0, FL)],
                                    list_hbm.at[pl.ds(off, FL)])
                    nleft = lptr - FL

                    def mv(k, _):
                        lbuf[pl.ds(k * 16, 16)] = \
                            lbuf[pl.ds(FL + k * 16, 16)]
                        return 0
                    lax.fori_loop(0, (nleft + 15) // 16, mv, 0)
                    return nleft, app + FL

                return lax.cond(lptr >= FL, do_flush, lambda a: a,
                                (lptr, app))

            def process(b, ph, carry):
                lptr, app = carry
                pltpu.make_async_copy(vtx_hbm.at[pl.ds(0, BS)], vb[ph],
                                      semi[ph]).wait()
                pltpu.make_async_copy(edg_hbm.at[pl.ds(0, BS)], eb[ph],
                                      semi[ph]).wait()

                @pl.when(b + 1 < NB)
                def _():
                    prefetch(b + 1, 1 - ph)

                def inner(j, lptr):
                    ev = eb[ph][pl.ds(j * 16, 16)]
                    vv = vb[ph][pl.ds(j * 16, 16)]
                    seg = ev if stage1 else vv
                    oth = vv if stage1 else ev
                    m = (seg >= lo) & (seg < hi)
                    p = ((seg - lo) << shift) | oth
                    plsc.store_compressed(lbuf.at[pl.ds(lptr, 16)], p,
                                          mask=m)
                    pc = plsc.all_reduce_population_count(m)
                    return lptr + pc[0]
                lptr = lax.fori_loop(0, BS // 16, inner, lptr, unroll=4)
                return flush_if_full((lptr, app))

            def pair(bp, carry):
                for ph in range(2):
                    b = bp * 2 + ph
                    carry = process(b, ph, carry)
                return carry

            lptr, app = lax.fori_loop(0, NB // 2, pair,
                                      (jnp.int32(0), jnp.int32(0)))
            off = pl.multiple_of(lbase + app, FL)
            pltpu.sync_copy(lbuf.at[pl.ds(0, FL)],
                            list_hbm.at[pl.ds(off, FL)])
            return app + lptr

        def replay(stage1, count_cnt, n, coff, src_hbm):
            shift = S1_SHIFT if stage1 else S2_SHIFT
            mask = S1_MASK if stage1 else S2_MASK
            nb = lax.div(n + RB - 1, RB)

            def prefetch_list(b, ph):
                off = pl.multiple_of(lbase + b * RB, RB)
                pltpu.async_copy(list_hbm.at[pl.ds(off, RB)],
                                 rlb[ph], seml[ph])

            @pl.when(nb > 0)
            def _():
                prefetch_list(0, 0)

            def accum(ph):
                def acc_body(r, _):
                    a = ca[ph][pl.ds(r, 16)][0]
                    for k in range(CW // 16):
                        sl = pl.ds(k * 16, 16)
                        acc[a, sl] = acc[a, sl] + rows[ph][r, sl]
                    if count_cnt:
                        base = (a // 16) * 16
                        cv = cnt[pl.ds(base, 16)]
                        cnt[pl.ds(base, 16)] = cv + jnp.where(
                            iota16 == (a - base), 1.0, 0.0)
                    return 0
                lax.fori_loop(0, RB, acc_body, 0, unroll=4)

            def wait_gather(ph):
                pltpu.make_async_copy(src_hbm.at[ci[ph]], rows[ph],
                                      semg[ph]).wait()

            def process(b, ph):
                pltpu.make_async_copy(list_hbm.at[pl.ds(0, RB)], rlb[ph],
                                      seml[ph]).wait()

                @pl.when(b + 1 < nb)
                def _():
                    prefetch_list(b + 1, 1 - ph)

                @plsc.parallel_loop(0, RB // 16, unroll=4)
                def _(j):
                    sl = pl.ds(j * 16, 16)
                    gi = b * RB + j * 16 + iota16
                    valid = gi < n
                    p = rlb[ph][sl]
                    seg = jnp.where(valid, p >> shift, DUMP)
                    gidx = jnp.where(valid, (p & mask) + coff,
                                     j * 16 + iota16)
                    ca[ph][sl] = seg
                    ci[ph][sl] = gidx

                pltpu.async_copy(src_hbm.at[ci[ph]], rows[ph], semg[ph])

                @pl.when(b > 0)
                def _():
                    wait_gather(1 - ph)
                    accum(1 - ph)

            def pair(bp, _):
                for ph in range(2):
                    b = bp * 2 + ph

                    @pl.when(b < nb)
                    def _():
                        process(b, ph)
                return 0

            lax.fori_loop(0, lax.div(nb + 1, 2), pair, 0)

            @pl.when(nb > 0)
            def _():
                @pl.when(lax.rem(nb, 2) == 1)
                def _():
                    wait_gather(0)
                    accum(0)

                @pl.when(lax.rem(nb, 2) == 0)
                def _():
                    wait_gather(1)
                    accum(1)

        n1 = scan(True, sid * EPT)
        zero_cnt()
        for t in range(2):
            c = cid * 2 + t
            zero_acc()
            replay(True, t == 0, n1, c * N, xwc_hbm)

            if t == 0:
                @plsc.parallel_loop(0, 1280 // 16, unroll=4)
                def _(j):
                    sl = pl.ds(j * 16, 16)
                    cnt[sl] = 1.0 / jnp.maximum(cnt[sl], 1.0)

            @plsc.parallel_loop(0, EPT, unroll=4)
            def _(e):
                inv = cnt[pl.ds(e, 16)][0]
                for k in range(CW // 16):
                    sl = pl.ds(k * 16, 16)
                    acc[e, sl] = acc[e, sl] * inv

            pltpu.sync_copy(acc.at[pl.ds(0, EPT)],
                            xec_hbm.at[pl.ds(c * E + sid * EPT, EPT)])

        plsc.subcore_barrier()

        n2 = scan(False, sid * VPT)
        for t in range(2):
            c = cid * 2 + t
            zero_acc()
            replay(False, False, n2, c * E, xec_hbm)
            pltpu.sync_copy(acc.at[pl.ds(0, VPT)],
                            xv_hbm.at[pl.ds(sid * VPT, VPT),
                                      pl.ds(c * CW, CW)])

    return conv(xwc, vertex, edges)


def kernel(X, vertex, edges, W):
    Xw, xwc = _matmul(X, W)
    Xv, _, _ = _sc_conv(xwc, vertex, edges)
    return _epilogue(Xw, Xv)
